# baseline (speedup 1.0000x reference)
import functools

import jax
import jax.numpy as jnp
from jax import lax
from jax.experimental import pallas as pl
from jax.experimental.pallas import tpu as pltpu
from jax.experimental.pallas import tpu_sc as plsc

_NE = 1024
_ED = 64
_BETA = 0.25
_TOK = 1024


def _k12_body(x_ref, w1b_ref, b1b_ref, w2_ref, b2_ref, pq_ref, o_ref):
    xb = x_ref[0].reshape(192, 512)
    pq = pq_ref[...]
    xg = [jnp.dot(xb[:, 128 * g:128 * g + 128], pq,
                  preferred_element_type=jnp.float32) for g in range(4)]
    xdx = [jnp.concatenate([xg[g][:, 32 * dx:32 * dx + 32]
                            for g in range(4)], axis=1)
           for dx in range(4)]
    for i in range(4):
        xs = jnp.concatenate([
            jnp.concatenate([xdx[dx][64 * c + 16 * i:64 * c + 16 * i + 16, :]
                             for c in range(3)], axis=0)
            for dx in range(4)], axis=0)
        ht = jnp.dot(w1b_ref[...], xs,
                     preferred_element_type=jnp.float32) + b1b_ref[...]
        ht = jnp.maximum(ht, 0.0)
        htp = jnp.dot(ht, pq, preferred_element_type=jnp.float32)
        b = jnp.concatenate([htp[:, 32 * d:32 * d + 32] for d in range(4)],
                            axis=0)
        h2t = jnp.dot(w2_ref[...], b,
                      preferred_element_type=jnp.float32) + b2_ref[...]
        o_ref[32 * i:32 * i + 32, :] = jnp.transpose(h2t)


def _shift_tokens(hr, dy, dx):
    s = 32 * dy + dx
    if s > 0:
        sh = jnp.concatenate(
            [hr[s:], jnp.zeros((s, hr.shape[1]), jnp.float32)], axis=0)
    elif s < 0:
        sh = jnp.concatenate(
            [jnp.zeros((-s, hr.shape[1]), jnp.float32), hr[:1024 + s]], axis=0)
    else:
        sh = hr
    if dx != 0:
        xo = jax.lax.broadcasted_iota(jnp.int32, (1024, 1), 0) % 32
        valid = (xo + dx >= 0) & (xo + dx < 32)
        sh = jnp.where(valid, sh, 0.0)
    return sh


def _k2_body(h_ref, wr1_ref, wr2_ref, wp_ref, bp_ref,
             ct_ref, csq_ref,
             zf_ref, idx_ref, perp_ref,
             cnt_ref):
    n = pl.program_id(0)

    h = h_ref[...]

    for _ in range(2):
        hr = jnp.maximum(h, 0.0)
        acc = jnp.zeros((1024, 64), jnp.float32)
        k = 0
        for ky in range(3):
            for kx in range(3):
                sh = _shift_tokens(hr, ky - 1, kx - 1)
                acc = acc + jnp.dot(sh, wr1_ref[k],
                                    preferred_element_type=jnp.float32)
                k += 1
        r = jnp.dot(jnp.maximum(acc, 0.0), wr2_ref[...],
                    preferred_element_type=jnp.float32)
        h = h + r

    h = jnp.maximum(h, 0.0)
    zf = jnp.dot(h, wp_ref[...],
                 preferred_element_type=jnp.float32) + bp_ref[...]

    scores = csq_ref[...] - 2.0 * jnp.dot(
        zf, ct_ref[...], preferred_element_type=jnp.float32)
    m = jnp.min(scores, axis=1, keepdims=True)
    jj = jax.lax.broadcasted_iota(jnp.int32, (1024, _NE), 1)
    idx = jnp.min(jnp.where(scores <= m, jj, _NE), axis=1, keepdims=True)

    onehot = (jj == idx).astype(jnp.float32)

    idx_ref[...] = idx[None]
    zf_ref[...] = zf

    @pl.when(n == 0)
    def _():
        cnt_ref[...] = jnp.zeros_like(cnt_ref)

    cnt_ref[...] += jnp.sum(onehot, axis=0, keepdims=True)

    @pl.when(n == pl.num_programs(0) - 1)
    def _():
        e_mean = cnt_ref[...] / (8.0 * _TOK)
        ent = jnp.sum(e_mean * jnp.log(e_mean + 1e-10))
        perp_ref[...] = jnp.full((1, 1), jnp.exp(-ent), jnp.float32)


_SC_CORES = 2
_SC_SUBCORES = 16
_NW = _SC_CORES * _SC_SUBCORES
_BPW = 8 * _TOK // _NW


def _sc_gather_body(table_hbm, idx_hbm, out_hbm, idx_v, rows_v, sem):
    wid = lax.axis_index("s") * _SC_CORES + lax.axis_index("c")
    base = wid * _BPW
    pltpu.sync_copy(idx_hbm.at[pl.ds(base, _BPW)], idx_v)
    pltpu.async_copy(table_hbm.at[idx_v], rows_v, sem).wait()
    pltpu.sync_copy(rows_v, out_hbm.at[pl.ds(base, _BPW)])


def _k3_body(zq_ref, zf_ref, zqo_ref, loss_ref, sse_ref):
    n = pl.program_id(0)
    zq1 = zq_ref[:, :_ED]
    zqo_ref[...] = jnp.transpose(zq1)[None]

    @pl.when(n == 0)
    def _():
        sse_ref[0, 0] = 0.0

    sse_ref[0, 0] += jnp.sum((zq1 - zf_ref[...]) ** 2)

    @pl.when(n == pl.num_programs(0) - 1)
    def _():
        loss_ref[...] = jnp.full(
            (1, 1),
            (1.0 + _BETA) * sse_ref[0, 0] / (8.0 * _TOK * _ED), jnp.float32)


def kernel(x, conv1_w, conv1_b, conv2_w, conv2_b, res_w1, res_w2,
           preq_w, preq_b, codebook):
    f32 = jnp.float32

    w1b = jnp.zeros((4, 64, 4, 3, 16), f32)
    w1t = conv1_w.transpose(0, 3, 1, 2)
    for y in range(4):
        w1b = w1b.at[y, :, :, :, 4 * y:4 * y + 4].set(w1t)
    w1b = w1b.reshape(256, 192)
    b1b = jnp.tile(conv1_b, 4).reshape(256, 1)
    w2p = conv2_w.transpose(0, 3, 2, 1).reshape(128, 1024)
    b2p = conv2_b.reshape(128, 1)
    ll = jnp.arange(128)
    pq = jnp.zeros((128, 128), f32).at[ll, 32 * (ll % 4) + ll // 4].set(1.0)

    h2 = pl.pallas_call(
        _k12_body,
        grid=(8, 8),
        in_specs=[
            pl.BlockSpec((1, 3, 64, 512), lambda n, i: (n, 0, i, 0)),
            pl.BlockSpec((256, 192), lambda n, i: (0, 0)),
            pl.BlockSpec((256, 1), lambda n, i: (0, 0)),
            pl.BlockSpec((128, 1024), lambda n, i: (0, 0)),
            pl.BlockSpec((128, 1), lambda n, i: (0, 0)),
            pl.BlockSpec((128, 128), lambda n, i: (0, 0)),
        ],
        out_specs=pl.BlockSpec((128, 128), lambda n, i: (n * 8 + i, 0)),
        out_shape=jax.ShapeDtypeStruct((8192, 128), f32),
    )(x, w1b, b1b, w2p, b2p, pq)

    wr1 = res_w1.transpose(2, 3, 1, 0).reshape(9, 128, 64)
    wr2 = res_w2.reshape(128, 64).T
    wp = preq_w.reshape(64, 128).T
    bp = preq_b.reshape(1, 64)
    ct = codebook.T
    csq = jnp.sum(codebook ** 2, axis=1).reshape(1, _NE)

    zf, idx, perp = pl.pallas_call(
        _k2_body,
        grid=(8,),
        in_specs=[
            pl.BlockSpec((_TOK, 128), lambda n: (n, 0)),
            pl.BlockSpec((9, 128, 64), lambda n: (0, 0, 0)),
            pl.BlockSpec((64, 128), lambda n: (0, 0)),
            pl.BlockSpec((128, 64), lambda n: (0, 0)),
            pl.BlockSpec((1, 64), lambda n: (0, 0)),
            pl.BlockSpec((_ED, _NE), lambda n: (0, 0)),
            pl.BlockSpec((1, _NE), lambda n: (0, 0)),
        ],
        out_specs=[
            pl.BlockSpec((_TOK, _ED), lambda n: (n, 0)),
            pl.BlockSpec((1, _TOK, 1), lambda n: (n, 0, 0)),
            pl.BlockSpec((1, 1), lambda n: (0, 0)),
        ],
        out_shape=[
            jax.ShapeDtypeStruct((8 * _TOK, _ED), f32),
            jax.ShapeDtypeStruct((8, _TOK, 1), jnp.int32),
            jax.ShapeDtypeStruct((1, 1), f32),
        ],
        scratch_shapes=[
            pltpu.VMEM((1, _NE), f32),
        ],
    )(h2, wr1, wr2, wp, bp, ct, csq)

    idx_flat = idx.reshape(8 * _TOK)

    sc_gather = functools.partial(
        pl.kernel,
        mesh=plsc.VectorSubcoreMesh(core_axis_name="c", subcore_axis_name="s"),
        out_type=jax.ShapeDtypeStruct((8 * _TOK, 128), f32),
        scratch_types=[
            pltpu.VMEM((_BPW,), jnp.int32),
            pltpu.VMEM((_BPW, 128), f32),
            pltpu.SemaphoreType.DMA,
        ],
    )(_sc_gather_body)
    cb_pad = jnp.pad(codebook, ((0, 0), (0, 128 - _ED)))
    zq1 = sc_gather(cb_pad, idx_flat)

    zq, loss = pl.pallas_call(
        _k3_body,
        grid=(8,),
        in_specs=[
            pl.BlockSpec((_TOK, 128), lambda n: (n, 0)),
            pl.BlockSpec((_TOK, _ED), lambda n: (n, 0)),
        ],
        out_specs=[
            pl.BlockSpec((1, _ED, _TOK), lambda n: (n, 0, 0)),
            pl.BlockSpec((1, 1), lambda n: (0, 0)),
        ],
        out_shape=[
            jax.ShapeDtypeStruct((8, _ED, _TOK), f32),
            jax.ShapeDtypeStruct((1, 1), f32),
        ],
        scratch_shapes=[
            pltpu.SMEM((1, 1), f32),
        ],
    )(zq1, zf)

    z_q = zq.reshape(8, _ED, 32, 32)
    idx_out = idx.reshape(8 * _TOK, 1)
    return (loss[0, 0], z_q, perp[0, 0], codebook, idx_out)


# speedup vs baseline: 1.0030x; 1.0030x over previous
import functools

import jax
import jax.numpy as jnp
from jax import lax
from jax.experimental import pallas as pl
from jax.experimental.pallas import tpu as pltpu
from jax.experimental.pallas import tpu_sc as plsc

_NE = 1024
_ED = 64
_BETA = 0.25
_TOK = 1024


def _k12_body(x_ref, w1b_ref, b1b_ref, w2_ref, b2_ref, pq_ref, o_ref):
    xb = x_ref[0].reshape(192, 512)
    pq = pq_ref[...]
    xg = [jnp.dot(xb[:, 128 * g:128 * g + 128], pq,
                  preferred_element_type=jnp.float32) for g in range(4)]
    xdx = [jnp.concatenate([xg[g][:, 32 * dx:32 * dx + 32]
                            for g in range(4)], axis=1)
           for dx in range(4)]
    for i in range(4):
        xs = jnp.concatenate([
            jnp.concatenate([xdx[dx][64 * c + 16 * i:64 * c + 16 * i + 16, :]
                             for c in range(3)], axis=0)
            for dx in range(4)], axis=0)
        ht = jnp.dot(w1b_ref[...], xs,
                     preferred_element_type=jnp.float32) + b1b_ref[...]
        ht = jnp.maximum(ht, 0.0)
        htp = jnp.dot(ht, pq, preferred_element_type=jnp.float32)
        b = jnp.concatenate([htp[:, 32 * d:32 * d + 32] for d in range(4)],
                            axis=0)
        h2t = jnp.dot(w2_ref[...], b,
                      preferred_element_type=jnp.float32) + b2_ref[...]
        o_ref[32 * i:32 * i + 32, :] = jnp.transpose(h2t)


def _shift_tokens(hr, dy, dx):
    s = 32 * dy + dx
    if s > 0:
        sh = jnp.concatenate(
            [hr[s:], jnp.zeros((s, hr.shape[1]), jnp.float32)], axis=0)
    elif s < 0:
        sh = jnp.concatenate(
            [jnp.zeros((-s, hr.shape[1]), jnp.float32), hr[:1024 + s]], axis=0)
    else:
        sh = hr
    if dx != 0:
        xo = jax.lax.broadcasted_iota(jnp.int32, (1024, 1), 0) % 32
        valid = (xo + dx >= 0) & (xo + dx < 32)
        sh = jnp.where(valid, sh, 0.0)
    return sh


def _k2_body(h_ref, wr1_ref, wr2_ref, wp_ref, bp_ref,
             ct_ref, csq_ref,
             zf_ref, idx_ref, perp_ref,
             cnt_ref):
    n = pl.program_id(0)

    h = h_ref[...]

    for _ in range(2):
        hr = jnp.maximum(h, 0.0)
        acc = jnp.zeros((1024, 64), jnp.float32)
        k = 0
        for ky in range(3):
            for kx in range(3):
                sh = _shift_tokens(hr, ky - 1, kx - 1)
                acc = acc + jnp.dot(sh, wr1_ref[k],
                                    preferred_element_type=jnp.float32)
                k += 1
        r = jnp.dot(jnp.maximum(acc, 0.0), wr2_ref[...],
                    preferred_element_type=jnp.float32)
        h = h + r

    h = jnp.maximum(h, 0.0)
    zf = jnp.dot(h, wp_ref[...],
                 preferred_element_type=jnp.float32) + bp_ref[...]

    scores = csq_ref[...] - 2.0 * jnp.dot(
        zf, ct_ref[...], preferred_element_type=jnp.float32)
    m = jnp.min(scores, axis=1, keepdims=True)
    jj = jax.lax.broadcasted_iota(jnp.int32, (1024, _NE), 1)
    idx = jnp.min(jnp.where(scores <= m, jj, _NE), axis=1, keepdims=True)

    onehot = (jj == idx).astype(jnp.float32)

    idx_ref[...] = idx[None]
    zf_ref[...] = zf

    @pl.when(n == 0)
    def _():
        cnt_ref[...] = jnp.zeros_like(cnt_ref)

    cnt_ref[...] += jnp.sum(onehot, axis=0, keepdims=True)

    @pl.when(n == pl.num_programs(0) - 1)
    def _():
        e_mean = cnt_ref[...] / (8.0 * _TOK)
        ent = jnp.sum(e_mean * jnp.log(e_mean + 1e-10))
        perp_ref[...] = jnp.full((1, 1), jnp.exp(-ent), jnp.float32)


_SC_CORES = 2
_SC_SUBCORES = 16
_NW = _SC_CORES * _SC_SUBCORES
_BPW = 8 * _TOK // _NW


_SC_CHUNKS = 8
_CPW = _BPW // _SC_CHUNKS


def _sc_gather_body(table_hbm, idx_hbm, out_hbm, *rest):
    idx_vs = rest[:_SC_CHUNKS]
    rows_v = rest[_SC_CHUNKS]
    sems = rest[_SC_CHUNKS + 1:]
    wid = lax.axis_index("s") * _SC_CORES + lax.axis_index("c")
    base = wid * _BPW
    copies = []
    for k in range(_SC_CHUNKS):
        pltpu.sync_copy(idx_hbm.at[pl.ds(base + k * _CPW, _CPW)], idx_vs[k])
        copies.append(pltpu.async_copy(
            table_hbm.at[idx_vs[k]],
            rows_v.at[pl.ds(k * _CPW, _CPW)], sems[k]))
    for c in copies:
        c.wait()
    pltpu.sync_copy(rows_v, out_hbm.at[pl.ds(base, _BPW)])


def _k3_body(zq_ref, zf_ref, zqo_ref, loss_ref, sse_ref):
    n = pl.program_id(0)
    zq1 = zq_ref[:, :_ED]
    zqo_ref[...] = jnp.transpose(zq1)[None]

    @pl.when(n == 0)
    def _():
        sse_ref[0, 0] = 0.0

    sse_ref[0, 0] += jnp.sum((zq1 - zf_ref[...]) ** 2)

    @pl.when(n == pl.num_programs(0) - 1)
    def _():
        loss_ref[...] = jnp.full(
            (1, 1),
            (1.0 + _BETA) * sse_ref[0, 0] / (8.0 * _TOK * _ED), jnp.float32)


def kernel(x, conv1_w, conv1_b, conv2_w, conv2_b, res_w1, res_w2,
           preq_w, preq_b, codebook):
    f32 = jnp.float32

    w1b = jnp.zeros((4, 64, 4, 3, 16), f32)
    w1t = conv1_w.transpose(0, 3, 1, 2)
    for y in range(4):
        w1b = w1b.at[y, :, :, :, 4 * y:4 * y + 4].set(w1t)
    w1b = w1b.reshape(256, 192)
    b1b = jnp.tile(conv1_b, 4).reshape(256, 1)
    w2p = conv2_w.transpose(0, 3, 2, 1).reshape(128, 1024)
    b2p = conv2_b.reshape(128, 1)
    ll = jnp.arange(128)
    pq = jnp.zeros((128, 128), f32).at[ll, 32 * (ll % 4) + ll // 4].set(1.0)

    h2 = pl.pallas_call(
        _k12_body,
        grid=(8, 8),
        in_specs=[
            pl.BlockSpec((1, 3, 64, 512), lambda n, i: (n, 0, i, 0)),
            pl.BlockSpec((256, 192), lambda n, i: (0, 0)),
            pl.BlockSpec((256, 1), lambda n, i: (0, 0)),
            pl.BlockSpec((128, 1024), lambda n, i: (0, 0)),
            pl.BlockSpec((128, 1), lambda n, i: (0, 0)),
            pl.BlockSpec((128, 128), lambda n, i: (0, 0)),
        ],
        out_specs=pl.BlockSpec((128, 128), lambda n, i: (n * 8 + i, 0)),
        out_shape=jax.ShapeDtypeStruct((8192, 128), f32),
    )(x, w1b, b1b, w2p, b2p, pq)

    wr1 = res_w1.transpose(2, 3, 1, 0).reshape(9, 128, 64)
    wr2 = res_w2.reshape(128, 64).T
    wp = preq_w.reshape(64, 128).T
    bp = preq_b.reshape(1, 64)
    ct = codebook.T
    csq = jnp.sum(codebook ** 2, axis=1).reshape(1, _NE)

    zf, idx, perp = pl.pallas_call(
        _k2_body,
        grid=(8,),
        in_specs=[
            pl.BlockSpec((_TOK, 128), lambda n: (n, 0)),
            pl.BlockSpec((9, 128, 64), lambda n: (0, 0, 0)),
            pl.BlockSpec((64, 128), lambda n: (0, 0)),
            pl.BlockSpec((128, 64), lambda n: (0, 0)),
            pl.BlockSpec((1, 64), lambda n: (0, 0)),
            pl.BlockSpec((_ED, _NE), lambda n: (0, 0)),
            pl.BlockSpec((1, _NE), lambda n: (0, 0)),
        ],
        out_specs=[
            pl.BlockSpec((_TOK, _ED), lambda n: (n, 0)),
            pl.BlockSpec((1, _TOK, 1), lambda n: (n, 0, 0)),
            pl.BlockSpec((1, 1), lambda n: (0, 0)),
        ],
        out_shape=[
            jax.ShapeDtypeStruct((8 * _TOK, _ED), f32),
            jax.ShapeDtypeStruct((8, _TOK, 1), jnp.int32),
            jax.ShapeDtypeStruct((1, 1), f32),
        ],
        scratch_shapes=[
            pltpu.VMEM((1, _NE), f32),
        ],
    )(h2, wr1, wr2, wp, bp, ct, csq)

    idx_flat = idx.reshape(8 * _TOK)

    sc_gather = functools.partial(
        pl.kernel,
        mesh=plsc.VectorSubcoreMesh(core_axis_name="c", subcore_axis_name="s"),
        out_type=jax.ShapeDtypeStruct((8 * _TOK, 128), f32),
        scratch_types=(
            [pltpu.VMEM((_CPW,), jnp.int32) for _ in range(_SC_CHUNKS)]
            + [pltpu.VMEM((_BPW, 128), f32)]
            + [pltpu.SemaphoreType.DMA for _ in range(_SC_CHUNKS)]
        ),
    )(_sc_gather_body)
    cb_pad = jnp.pad(codebook, ((0, 0), (0, 128 - _ED)))
    zq1 = sc_gather(cb_pad, idx_flat)

    zq, loss = pl.pallas_call(
        _k3_body,
        grid=(8,),
        in_specs=[
            pl.BlockSpec((_TOK, 128), lambda n: (n, 0)),
            pl.BlockSpec((_TOK, _ED), lambda n: (n, 0)),
        ],
        out_specs=[
            pl.BlockSpec((1, _ED, _TOK), lambda n: (n, 0, 0)),
            pl.BlockSpec((1, 1), lambda n: (0, 0)),
        ],
        out_shape=[
            jax.ShapeDtypeStruct((8, _ED, _TOK), f32),
            jax.ShapeDtypeStruct((1, 1), f32),
        ],
        scratch_shapes=[
            pltpu.SMEM((1, 1), f32),
        ],
    )(zq1, zf)

    z_q = zq.reshape(8, _ED, 32, 32)
    idx_out = idx.reshape(8 * _TOK, 1)
    return (loss[0, 0], z_q, perp[0, 0], codebook, idx_out)


# speedup vs baseline: 2.4571x; 2.4497x over previous
import jax
import jax.numpy as jnp
from jax.experimental import pallas as pl
from jax.experimental.pallas import tpu as pltpu

_NE = 1024
_ED = 64
_BETA = 0.25
_TOK = 1024


def _k12_body(x_ref, w1b_ref, b1b_ref, w2_ref, b2_ref, pq_ref, o_ref):
    xb = x_ref[0].reshape(192, 512).astype(jnp.bfloat16)
    pq = pq_ref[...]
    xg = [jnp.dot(xb[:, 128 * g:128 * g + 128], pq,
                  preferred_element_type=jnp.float32).astype(jnp.bfloat16)
          for g in range(4)]
    xdx = [jnp.concatenate([xg[g][:, 32 * dx:32 * dx + 32]
                            for g in range(4)], axis=1)
           for dx in range(4)]
    for i in range(4):
        xs = jnp.concatenate([
            jnp.concatenate([xdx[dx][64 * c + 16 * i:64 * c + 16 * i + 16, :]
                             for c in range(3)], axis=0)
            for dx in range(4)], axis=0)
        ht = jnp.dot(w1b_ref[...], xs,
                     preferred_element_type=jnp.float32) + b1b_ref[...]
        ht = jnp.maximum(ht, 0.0).astype(jnp.bfloat16)
        htp = jnp.dot(ht, pq,
                      preferred_element_type=jnp.float32).astype(jnp.bfloat16)
        b = jnp.concatenate([htp[:, 32 * d:32 * d + 32] for d in range(4)],
                            axis=0)
        h2t = jnp.dot(w2_ref[...], b,
                      preferred_element_type=jnp.float32) + b2_ref[...]
        o_ref[32 * i:32 * i + 32, :] = jnp.transpose(h2t)


def _shift_tokens(hr, dy, dx):
    s = 32 * dy + dx
    if s > 0:
        sh = jnp.concatenate(
            [hr[s:], jnp.zeros((s, hr.shape[1]), jnp.float32)], axis=0)
    elif s < 0:
        sh = jnp.concatenate(
            [jnp.zeros((-s, hr.shape[1]), jnp.float32), hr[:1024 + s]], axis=0)
    else:
        sh = hr
    if dx != 0:
        xo = jax.lax.broadcasted_iota(jnp.int32, (1024, 1), 0) % 32
        valid = (xo + dx >= 0) & (xo + dx < 32)
        sh = jnp.where(valid, sh, 0.0)
    return sh


def _k2_body(h_ref, wr1_ref, wr2_ref, wp_ref, bp_ref,
             c_ref, ct_ref, csq_ref,
             zq_ref, idx_ref, loss_ref, perp_ref,
             cnt_ref, sse_ref):
    n = pl.program_id(0)

    h = h_ref[...]

    for _ in range(2):
        hr = jnp.maximum(h, 0.0)
        acc = jnp.zeros((1024, 64), jnp.float32)
        k = 0
        for ky in range(3):
            for kx in range(3):
                sh = _shift_tokens(hr, ky - 1, kx - 1)
                acc = acc + jnp.dot(sh, wr1_ref[k],
                                    preferred_element_type=jnp.float32)
                k += 1
        r = jnp.dot(jnp.maximum(acc, 0.0), wr2_ref[...],
                    preferred_element_type=jnp.float32)
        h = h + r

    h = jnp.maximum(h, 0.0)
    zf = jnp.dot(h, wp_ref[...],
                 preferred_element_type=jnp.float32) + bp_ref[...]

    scores = csq_ref[...] - 2.0 * jnp.dot(
        zf.astype(jnp.bfloat16), ct_ref[...].astype(jnp.bfloat16),
        preferred_element_type=jnp.float32)
    m = jnp.min(scores, axis=1, keepdims=True)
    jj = jax.lax.broadcasted_iota(jnp.int32, (1024, _NE), 1)
    idx = jnp.min(jnp.where(scores <= m, jj, _NE), axis=1, keepdims=True)

    onehot = (jj == idx).astype(jnp.float32)
    zq1 = jnp.dot(onehot, c_ref[...], preferred_element_type=jnp.float32)

    idx_ref[...] = idx[None]
    zq_ref[...] = jnp.transpose(zq1)[None]

    @pl.when(n == 0)
    def _():
        cnt_ref[...] = jnp.zeros_like(cnt_ref)
        sse_ref[0, 0] = 0.0

    cnt_ref[...] += jnp.sum(onehot, axis=0, keepdims=True)
    sse_ref[0, 0] += jnp.sum((zq1 - zf) ** 2)

    @pl.when(n == pl.num_programs(0) - 1)
    def _():
        total = sse_ref[0, 0]
        loss_ref[...] = jnp.full(
            (1, 1), (1.0 + _BETA) * total / (8.0 * _TOK * _ED), jnp.float32)
        e_mean = cnt_ref[...] / (8.0 * _TOK)
        ent = jnp.sum(e_mean * jnp.log(e_mean + 1e-10))
        perp_ref[...] = jnp.full((1, 1), jnp.exp(-ent), jnp.float32)


def kernel(x, conv1_w, conv1_b, conv2_w, conv2_b, res_w1, res_w2,
           preq_w, preq_b, codebook):
    f32 = jnp.float32

    w1b = jnp.zeros((4, 64, 4, 3, 16), f32)
    w1t = conv1_w.transpose(0, 3, 1, 2)
    for y in range(4):
        w1b = w1b.at[y, :, :, :, 4 * y:4 * y + 4].set(w1t)
    w1b = w1b.reshape(256, 192).astype(jnp.bfloat16)
    b1b = jnp.tile(conv1_b, 4).reshape(256, 1)
    w2p = conv2_w.transpose(0, 3, 2, 1).reshape(128, 1024).astype(jnp.bfloat16)
    b2p = conv2_b.reshape(128, 1)
    ll = jnp.arange(128)
    pq = jnp.zeros((128, 128), jnp.bfloat16).at[
        ll, 32 * (ll % 4) + ll // 4].set(1.0)

    h2 = pl.pallas_call(
        _k12_body,
        grid=(8, 8),
        in_specs=[
            pl.BlockSpec((1, 3, 64, 512), lambda n, i: (n, 0, i, 0)),
            pl.BlockSpec((256, 192), lambda n, i: (0, 0)),
            pl.BlockSpec((256, 1), lambda n, i: (0, 0)),
            pl.BlockSpec((128, 1024), lambda n, i: (0, 0)),
            pl.BlockSpec((128, 1), lambda n, i: (0, 0)),
            pl.BlockSpec((128, 128), lambda n, i: (0, 0)),
        ],
        out_specs=pl.BlockSpec((128, 128), lambda n, i: (n * 8 + i, 0)),
        out_shape=jax.ShapeDtypeStruct((8192, 128), f32),
    )(x, w1b, b1b, w2p, b2p, pq)

    wr1 = res_w1.transpose(2, 3, 1, 0).reshape(9, 128, 64)
    wr2 = res_w2.reshape(128, 64).T
    wp = preq_w.reshape(64, 128).T
    bp = preq_b.reshape(1, 64)
    ct = codebook.T
    csq = jnp.sum(codebook ** 2, axis=1).reshape(1, _NE)

    zq, idx, loss, perp = pl.pallas_call(
        _k2_body,
        grid=(8,),
        in_specs=[
            pl.BlockSpec((_TOK, 128), lambda n: (n, 0)),
            pl.BlockSpec((9, 128, 64), lambda n: (0, 0, 0)),
            pl.BlockSpec((64, 128), lambda n: (0, 0)),
            pl.BlockSpec((128, 64), lambda n: (0, 0)),
            pl.BlockSpec((1, 64), lambda n: (0, 0)),
            pl.BlockSpec((_NE, _ED), lambda n: (0, 0)),
            pl.BlockSpec((_ED, _NE), lambda n: (0, 0)),
            pl.BlockSpec((1, _NE), lambda n: (0, 0)),
        ],
        out_specs=[
            pl.BlockSpec((1, _ED, _TOK), lambda n: (n, 0, 0)),
            pl.BlockSpec((1, _TOK, 1), lambda n: (n, 0, 0)),
            pl.BlockSpec((1, 1), lambda n: (0, 0)),
            pl.BlockSpec((1, 1), lambda n: (0, 0)),
        ],
        out_shape=[
            jax.ShapeDtypeStruct((8, _ED, _TOK), f32),
            jax.ShapeDtypeStruct((8, _TOK, 1), jnp.int32),
            jax.ShapeDtypeStruct((1, 1), f32),
            jax.ShapeDtypeStruct((1, 1), f32),
        ],
        scratch_shapes=[
            pltpu.VMEM((1, _NE), f32),
            pltpu.SMEM((1, 1), f32),
        ],
    )(h2, wr1, wr2, wp, bp, codebook, ct, csq)

    z_q = zq.reshape(8, _ED, 32, 32)
    idx_out = idx.reshape(8 * _TOK, 1)
    return (loss[0, 0], z_q, perp[0, 0], codebook, idx_out)


# speedup vs baseline: 2.6743x; 1.0884x over previous
import jax
import jax.numpy as jnp
from jax.experimental import pallas as pl
from jax.experimental.pallas import tpu as pltpu

_NE = 1024
_ED = 64
_BETA = 0.25
_TOK = 1024


def _k12_body(x_ref, w1b_ref, b1b_ref, w2_ref, b2_ref, pq_ref, o_ref):
    xb = x_ref[0].reshape(192, 512).astype(jnp.bfloat16)
    pq = pq_ref[...]
    xg = [jnp.dot(xb[:, 128 * g:128 * g + 128], pq,
                  preferred_element_type=jnp.float32).astype(jnp.bfloat16)
          for g in range(4)]
    xs = jnp.concatenate([
        jnp.concatenate([xg[g][:, 32 * dx:32 * dx + 32] for g in range(4)],
                        axis=1)
        for dx in range(4)], axis=0)
    ht = jnp.dot(w1b_ref[...], xs,
                 preferred_element_type=jnp.float32) + b1b_ref[...]
    ht = jnp.maximum(ht, 0.0).astype(jnp.bfloat16)
    htp = jnp.dot(ht, pq,
                  preferred_element_type=jnp.float32).astype(jnp.bfloat16)
    ball = jnp.concatenate([
        jnp.concatenate([htp[256 * i:256 * i + 256, 32 * d:32 * d + 32]
                         for d in range(4)], axis=0)
        for i in range(4)], axis=1)
    h2t = jnp.dot(w2_ref[...], ball,
                  preferred_element_type=jnp.float32) + b2_ref[...]
    o_ref[...] = jnp.transpose(h2t)


def _shift_tokens(hr, dy, dx):
    s = 32 * dy + dx
    if s > 0:
        sh = jnp.concatenate(
            [hr[s:], jnp.zeros((s, hr.shape[1]), jnp.float32)], axis=0)
    elif s < 0:
        sh = jnp.concatenate(
            [jnp.zeros((-s, hr.shape[1]), jnp.float32), hr[:1024 + s]], axis=0)
    else:
        sh = hr
    if dx != 0:
        xo = jax.lax.broadcasted_iota(jnp.int32, (1024, 1), 0) % 32
        valid = (xo + dx >= 0) & (xo + dx < 32)
        sh = jnp.where(valid, sh, 0.0)
    return sh


def _k2_body(h_ref, wr1_ref, wr2_ref, wp_ref, bp_ref,
             c_ref, ct_ref, csq_ref,
             zq_ref, idx_ref, loss_ref, perp_ref,
             cnt_ref, sse_ref):
    n = pl.program_id(0)

    h = h_ref[...]

    for _ in range(2):
        hr = jnp.maximum(h, 0.0)
        acc = jnp.zeros((1024, 64), jnp.float32)
        k = 0
        for ky in range(3):
            for kx in range(3):
                sh = _shift_tokens(hr, ky - 1, kx - 1)
                acc = acc + jnp.dot(sh, wr1_ref[k],
                                    preferred_element_type=jnp.float32)
                k += 1
        r = jnp.dot(jnp.maximum(acc, 0.0), wr2_ref[...],
                    preferred_element_type=jnp.float32)
        h = h + r

    h = jnp.maximum(h, 0.0)
    zf = jnp.dot(h, wp_ref[...],
                 preferred_element_type=jnp.float32) + bp_ref[...]

    scores = csq_ref[...] - 2.0 * jnp.dot(
        zf.astype(jnp.bfloat16), ct_ref[...].astype(jnp.bfloat16),
        preferred_element_type=jnp.float32)
    m = jnp.min(scores, axis=1, keepdims=True)
    jj = jax.lax.broadcasted_iota(jnp.int32, (1024, _NE), 1)
    idx = jnp.min(jnp.where(scores <= m, jj, _NE), axis=1, keepdims=True)

    onehot = (jj == idx).astype(jnp.float32)
    zq1 = jnp.dot(onehot, c_ref[...], preferred_element_type=jnp.float32)

    idx_ref[...] = idx[None]
    zq_ref[...] = jnp.transpose(zq1)[None]

    @pl.when(n == 0)
    def _():
        cnt_ref[...] = jnp.zeros_like(cnt_ref)
        sse_ref[0, 0] = 0.0

    cnt_ref[...] += jnp.sum(onehot, axis=0, keepdims=True)
    sse_ref[0, 0] += jnp.sum((zq1 - zf) ** 2)

    @pl.when(n == pl.num_programs(0) - 1)
    def _():
        total = sse_ref[0, 0]
        loss_ref[...] = jnp.full(
            (1, 1), (1.0 + _BETA) * total / (8.0 * _TOK * _ED), jnp.float32)
        e_mean = cnt_ref[...] / (8.0 * _TOK)
        ent = jnp.sum(e_mean * jnp.log(e_mean + 1e-10))
        perp_ref[...] = jnp.full((1, 1), jnp.exp(-ent), jnp.float32)


def kernel(x, conv1_w, conv1_b, conv2_w, conv2_b, res_w1, res_w2,
           preq_w, preq_b, codebook):
    f32 = jnp.float32

    w1b = jnp.zeros((16, 64, 4, 3, 64), f32)
    w1t = conv1_w.transpose(0, 3, 1, 2)
    for y in range(16):
        w1b = w1b.at[y, :, :, :, 4 * y:4 * y + 4].set(w1t)
    w1b = w1b.reshape(1024, 768).astype(jnp.bfloat16)
    b1b = jnp.tile(conv1_b, 16).reshape(1024, 1)
    w2p = conv2_w.transpose(0, 3, 2, 1).reshape(128, 1024).astype(jnp.bfloat16)
    b2p = conv2_b.reshape(128, 1)
    ll = jnp.arange(128)
    pq = jnp.zeros((128, 128), jnp.bfloat16).at[
        ll, 32 * (ll % 4) + ll // 4].set(1.0)

    h2 = pl.pallas_call(
        _k12_body,
        grid=(8, 8),
        in_specs=[
            pl.BlockSpec((1, 3, 64, 512), lambda n, i: (n, 0, i, 0)),
            pl.BlockSpec((1024, 768), lambda n, i: (0, 0)),
            pl.BlockSpec((1024, 1), lambda n, i: (0, 0)),
            pl.BlockSpec((128, 1024), lambda n, i: (0, 0)),
            pl.BlockSpec((128, 1), lambda n, i: (0, 0)),
            pl.BlockSpec((128, 128), lambda n, i: (0, 0)),
        ],
        out_specs=pl.BlockSpec((128, 128), lambda n, i: (n * 8 + i, 0)),
        out_shape=jax.ShapeDtypeStruct((8192, 128), f32),
    )(x, w1b, b1b, w2p, b2p, pq)

    wr1 = res_w1.transpose(2, 3, 1, 0).reshape(9, 128, 64)
    wr2 = res_w2.reshape(128, 64).T
    wp = preq_w.reshape(64, 128).T
    bp = preq_b.reshape(1, 64)
    ct = codebook.T
    csq = jnp.sum(codebook ** 2, axis=1).reshape(1, _NE)

    zq, idx, loss, perp = pl.pallas_call(
        _k2_body,
        grid=(8,),
        in_specs=[
            pl.BlockSpec((_TOK, 128), lambda n: (n, 0)),
            pl.BlockSpec((9, 128, 64), lambda n: (0, 0, 0)),
            pl.BlockSpec((64, 128), lambda n: (0, 0)),
            pl.BlockSpec((128, 64), lambda n: (0, 0)),
            pl.BlockSpec((1, 64), lambda n: (0, 0)),
            pl.BlockSpec((_NE, _ED), lambda n: (0, 0)),
            pl.BlockSpec((_ED, _NE), lambda n: (0, 0)),
            pl.BlockSpec((1, _NE), lambda n: (0, 0)),
        ],
        out_specs=[
            pl.BlockSpec((1, _ED, _TOK), lambda n: (n, 0, 0)),
            pl.BlockSpec((1, _TOK, 1), lambda n: (n, 0, 0)),
            pl.BlockSpec((1, 1), lambda n: (0, 0)),
            pl.BlockSpec((1, 1), lambda n: (0, 0)),
        ],
        out_shape=[
            jax.ShapeDtypeStruct((8, _ED, _TOK), f32),
            jax.ShapeDtypeStruct((8, _TOK, 1), jnp.int32),
            jax.ShapeDtypeStruct((1, 1), f32),
            jax.ShapeDtypeStruct((1, 1), f32),
        ],
        scratch_shapes=[
            pltpu.VMEM((1, _NE), f32),
            pltpu.SMEM((1, 1), f32),
        ],
    )(h2, wr1, wr2, wp, bp, codebook, ct, csq)

    z_q = zq.reshape(8, _ED, 32, 32)
    idx_out = idx.reshape(8 * _TOK, 1)
    return (loss[0, 0], z_q, perp[0, 0], codebook, idx_out)


# speedup vs baseline: 2.9825x; 1.1152x over previous
import jax
import jax.numpy as jnp
from jax.experimental import pallas as pl
from jax.experimental.pallas import tpu as pltpu

_NE = 1024
_ED = 64
_BETA = 0.25
_TOK = 1024


def _k12_body(x_ref, w1b_ref, b1b_ref, w2_ref, b2_ref, pq_ref, o_ref):
    xb = x_ref[0].reshape(192, 512).astype(jnp.bfloat16)
    pq = pq_ref[...]
    xg = [jnp.dot(xb[:, 128 * g:128 * g + 128], pq,
                  preferred_element_type=jnp.float32).astype(jnp.bfloat16)
          for g in range(4)]
    xs = jnp.concatenate([
        jnp.concatenate([xg[g][:, 32 * dx:32 * dx + 32] for g in range(4)],
                        axis=1)
        for dx in range(4)], axis=0)
    ht = jnp.dot(w1b_ref[...], xs,
                 preferred_element_type=jnp.float32) + b1b_ref[...]
    ht = jnp.maximum(ht, 0.0).astype(jnp.bfloat16)
    htp = jnp.dot(ht, pq,
                  preferred_element_type=jnp.float32).astype(jnp.bfloat16)
    ball = jnp.concatenate([
        jnp.concatenate([htp[256 * i:256 * i + 256, 32 * d:32 * d + 32]
                         for d in range(4)], axis=0)
        for i in range(4)], axis=1)
    h2t = jnp.dot(w2_ref[...], ball,
                  preferred_element_type=jnp.float32) + b2_ref[...]
    o_ref[...] = jnp.transpose(h2t)


def _shift_tokens(hr, dy, dx):
    s = 32 * dy + dx
    if s > 0:
        sh = jnp.concatenate(
            [hr[s:], jnp.zeros((s, hr.shape[1]), jnp.float32)], axis=0)
    elif s < 0:
        sh = jnp.concatenate(
            [jnp.zeros((-s, hr.shape[1]), jnp.float32), hr[:1024 + s]], axis=0)
    else:
        sh = hr
    if dx != 0:
        xo = jax.lax.broadcasted_iota(jnp.int32, (1024, 1), 0) % 32
        valid = (xo + dx >= 0) & (xo + dx < 32)
        sh = jnp.where(valid, sh, 0.0)
    return sh


def _k2_body(h_ref, wr1_ref, wr2_ref, wp_ref, bp_ref,
             c_ref, ct_ref, csq_ref,
             zq_ref, idx_ref, loss_ref, perp_ref,
             cnt_ref, sse_ref):
    n = pl.program_id(0)

    h = h_ref[...]

    for _ in range(2):
        hr = jnp.maximum(h, 0.0)
        acc = jnp.zeros((1024, 64), jnp.float32)
        k = 0
        for ky in range(3):
            for kx in range(3):
                sh = _shift_tokens(hr, ky - 1, kx - 1)
                acc = acc + jnp.dot(sh, wr1_ref[k],
                                    preferred_element_type=jnp.float32)
                k += 1
        r = jnp.dot(jnp.maximum(acc, 0.0), wr2_ref[...],
                    preferred_element_type=jnp.float32)
        h = h + r

    h = jnp.maximum(h, 0.0)
    zf = jnp.dot(h, wp_ref[...],
                 preferred_element_type=jnp.float32) + bp_ref[...]

    scores = csq_ref[...] - 2.0 * jnp.dot(
        zf.astype(jnp.bfloat16), ct_ref[...].astype(jnp.bfloat16),
        preferred_element_type=jnp.float32)
    m = jnp.min(scores, axis=1, keepdims=True)
    jj = jax.lax.broadcasted_iota(jnp.int32, (1024, _NE), 1)
    idx = jnp.min(jnp.where(scores <= m, jj, _NE), axis=1, keepdims=True)

    onehot = (jj == idx).astype(jnp.float32)
    zq1 = jnp.dot(onehot, c_ref[...], preferred_element_type=jnp.float32)

    idx_ref[...] = idx[None]
    zq_ref[...] = jnp.transpose(zq1)[None]

    @pl.when(n == 0)
    def _():
        cnt_ref[...] = jnp.zeros_like(cnt_ref)
        sse_ref[0, 0] = 0.0

    cnt_ref[...] += jnp.sum(onehot, axis=0, keepdims=True)
    sse_ref[0, 0] += jnp.sum((zq1 - zf) ** 2)

    @pl.when(n == pl.num_programs(0) - 1)
    def _():
        total = sse_ref[0, 0]
        loss_ref[...] = jnp.full(
            (1, 1), (1.0 + _BETA) * total / (8.0 * _TOK * _ED), jnp.float32)
        e_mean = cnt_ref[...] / (8.0 * _TOK)
        ent = jnp.sum(e_mean * jnp.log(e_mean + 1e-10))
        perp_ref[...] = jnp.full((1, 1), jnp.exp(-ent), jnp.float32)


def kernel(x, conv1_w, conv1_b, conv2_w, conv2_b, res_w1, res_w2,
           preq_w, preq_b, codebook):
    f32 = jnp.float32

    w1t = conv1_w.transpose(0, 3, 1, 2)
    w1b = jnp.einsum('yb,odcr->yodcbr', jnp.eye(16, dtype=f32), w1t)
    w1b = w1b.reshape(1024, 768).astype(jnp.bfloat16)
    b1b = jnp.tile(conv1_b, 16).reshape(1024, 1)
    w2p = conv2_w.transpose(0, 3, 2, 1).reshape(128, 1024).astype(jnp.bfloat16)
    b2p = conv2_b.reshape(128, 1)
    ll = jnp.arange(128)
    pq = jnp.zeros((128, 128), jnp.bfloat16).at[
        ll, 32 * (ll % 4) + ll // 4].set(1.0)

    h2 = pl.pallas_call(
        _k12_body,
        grid=(8, 8),
        in_specs=[
            pl.BlockSpec((1, 3, 64, 512), lambda n, i: (n, 0, i, 0)),
            pl.BlockSpec((1024, 768), lambda n, i: (0, 0)),
            pl.BlockSpec((1024, 1), lambda n, i: (0, 0)),
            pl.BlockSpec((128, 1024), lambda n, i: (0, 0)),
            pl.BlockSpec((128, 1), lambda n, i: (0, 0)),
            pl.BlockSpec((128, 128), lambda n, i: (0, 0)),
        ],
        out_specs=pl.BlockSpec((128, 128), lambda n, i: (n * 8 + i, 0)),
        out_shape=jax.ShapeDtypeStruct((8192, 128), f32),
    )(x, w1b, b1b, w2p, b2p, pq)

    wr1 = res_w1.transpose(2, 3, 1, 0).reshape(9, 128, 64)
    wr2 = res_w2.reshape(128, 64).T
    wp = preq_w.reshape(64, 128).T
    bp = preq_b.reshape(1, 64)
    ct = codebook.T
    csq = jnp.sum(codebook ** 2, axis=1).reshape(1, _NE)

    zq, idx, loss, perp = pl.pallas_call(
        _k2_body,
        grid=(8,),
        in_specs=[
            pl.BlockSpec((_TOK, 128), lambda n: (n, 0)),
            pl.BlockSpec((9, 128, 64), lambda n: (0, 0, 0)),
            pl.BlockSpec((64, 128), lambda n: (0, 0)),
            pl.BlockSpec((128, 64), lambda n: (0, 0)),
            pl.BlockSpec((1, 64), lambda n: (0, 0)),
            pl.BlockSpec((_NE, _ED), lambda n: (0, 0)),
            pl.BlockSpec((_ED, _NE), lambda n: (0, 0)),
            pl.BlockSpec((1, _NE), lambda n: (0, 0)),
        ],
        out_specs=[
            pl.BlockSpec((1, _ED, _TOK), lambda n: (n, 0, 0)),
            pl.BlockSpec((1, _TOK, 1), lambda n: (n, 0, 0)),
            pl.BlockSpec((1, 1), lambda n: (0, 0)),
            pl.BlockSpec((1, 1), lambda n: (0, 0)),
        ],
        out_shape=[
            jax.ShapeDtypeStruct((8, _ED, _TOK), f32),
            jax.ShapeDtypeStruct((8, _TOK, 1), jnp.int32),
            jax.ShapeDtypeStruct((1, 1), f32),
            jax.ShapeDtypeStruct((1, 1), f32),
        ],
        scratch_shapes=[
            pltpu.VMEM((1, _NE), f32),
            pltpu.SMEM((1, 1), f32),
        ],
    )(h2, wr1, wr2, wp, bp, codebook, ct, csq)

    z_q = zq.reshape(8, _ED, 32, 32)
    idx_out = idx.reshape(8 * _TOK, 1)
    return (loss[0, 0], z_q, perp[0, 0], codebook, idx_out)
